# Initial kernel scaffold; baseline (speedup 1.0000x reference)
import jax
import jax.numpy as jnp
from jax.experimental import pallas as pl


def kernel(seq_word_ids, deno_labels, cono_labels, pretrained_table, deno_table, cono_table, W_deno, b_deno, W_cono, b_cono):
    raise NotImplementedError("write your pallas kernel here")


# speedup vs baseline: 16.2374x; 16.2374x over previous
import functools
import math

import jax
import jax.numpy as jnp
from jax import lax
from jax.experimental import pallas as pl
from jax.experimental.pallas import tpu as pltpu
from jax.experimental.pallas import tpu_sc as plsc

V = 100000
D = 128
B = 4096
L = 50
ND = 41
NCLS = 2
DQ = 128
DS = 96


_VB = 4000


def _precompute_body(deno_ref, cono_ref, pre_ref, w0_ref, w1_ref, q_ref):
    d = deno_ref[...]
    c = cono_ref[...]
    p = pre_ref[...]
    q = jnp.dot(d, w0_ref[...], preferred_element_type=jnp.float32)
    q = q + jnp.dot(c, w1_ref[...], preferred_element_type=jnp.float32)
    s = d + c
    num = jnp.sum(s * p, axis=1, keepdims=True)
    den = (jnp.sqrt(jnp.sum(s * s, axis=1, keepdims=True))
           * jnp.sqrt(jnp.sum(p * p, axis=1, keepdims=True)) + 1e-8)
    cos = num / den
    colq = lax.broadcasted_iota(jnp.int32, q.shape, 1)
    q_ref[...] = jnp.where(colq == 86, cos, q)


_NW = 32
_BPW = B // _NW
_CHB = 2
_CHT = _CHB * L
_NCH = _BPW // _CHB

@functools.lru_cache(maxsize=1)
def _make_gather_sum():
    mesh = plsc.VectorSubcoreMesh(core_axis_name="c", subcore_axis_name="s")

    nbuf = 4
    njc = DS // 16

    @functools.partial(
        pl.kernel,
        out_type=jax.ShapeDtypeStruct((B, DS), jnp.float32),
        mesh=mesh,
        scratch_types=[
            pltpu.VMEM((_NCH, _CHT), jnp.int32),
        ] + [pltpu.VMEM((_CHT, DQ), jnp.float32)] * nbuf + [
            pltpu.VMEM((_BPW, DS), jnp.float32),
        ] + [pltpu.SemaphoreType.DMA] * nbuf,
    )
    def _gather_sum(q_hbm, idx_hbm, out_hbm, idx_v, *scratch):
        bufs = scratch[:nbuf]
        out_v = scratch[nbuf]
        sems = scratch[nbuf + 1:]
        wid = lax.axis_index("s") * 2 + lax.axis_index("c")
        pltpu.sync_copy(idx_hbm.at[wid], idx_v)

        def fire(c, buf, sem):
            pltpu.async_copy(q_hbm.at[idx_v.at[c]], buf, sem)

        def wait(buf, sem):
            pltpu.make_async_copy(q_hbm.at[idx_v.at[0]], buf, sem).wait()

        def reduce_chunk(c, buf):
            for b in range(_CHB):
                base = b * L

                def lbody(l, accs):
                    return tuple(
                        accs[j] + buf[base + l, pl.ds(j * 16, 16)]
                        for j in range(njc))

                accs = lax.fori_loop(
                    0, L, lbody,
                    tuple(jnp.zeros((16,), jnp.float32)
                          for _ in range(njc)),
                    unroll=10)
                for j in range(njc):
                    out_v[c * _CHB + b, pl.ds(j * 16, 16)] = accs[j]

        for k in range(nbuf):
            fire(k, bufs[k], sems[k])

        nsup = _NCH // nbuf

        def body(s, carry):
            base = nbuf * s
            for k in range(nbuf):
                wait(bufs[k], sems[k])
                reduce_chunk(base + k, bufs[k])

                @pl.when(s < nsup - 1)
                def _(k=k):
                    fire(base + k + nbuf, bufs[k], sems[k])
            return carry

        lax.fori_loop(0, nsup, body, 0)
        pltpu.sync_copy(out_v, out_hbm.at[pl.ds(wid * _BPW, _BPW)])

    return _gather_sum


def _loss_body(s_ref, dl_ref, cl_ref, bias_ref, out_ref):
    S = s_ref[...]
    x = S * (1.0 / L) + bias_ref[...]
    col = lax.broadcasted_iota(jnp.int32, (B, DS), 1)
    dl = dl_ref[...]
    cl = cl_ref[...]

    def grp(lo, hi):
        mask = (col >= lo) & (col < hi)
        xm = jnp.where(mask, x, -1e30)
        m = jnp.max(xm, axis=1, keepdims=True)
        e = jnp.where(mask, jnp.exp(x - m), 0.0)
        logZ = jnp.log(jnp.sum(e, axis=1, keepdims=True))
        sum_x = jnp.sum(jnp.where(mask, x, 0.0))
        return m, logZ, sum_x

    def ce(lo, m, logZ, lab):
        sel = col == (lab + lo)
        x_sel = jnp.sum(jnp.where(sel, x, 0.0))
        return (jnp.sum(m) + jnp.sum(logZ) - x_sel) * (1.0 / B)

    def kl(gsize, m, logZ, sum_x):
        u = 1.0 / gsize
        lu = math.log(u)
        return u * (B * gsize * lu - sum_x
                    + gsize * (jnp.sum(m) + jnp.sum(logZ))) * (1.0 / B)

    m0, lz0, _ = grp(0, 41)
    m1, lz1, sx1 = grp(41, 43)
    m2, lz2, sx2 = grp(43, 84)
    m3, lz3, _ = grp(84, 86)

    ds_dp = ce(0, m0, lz0, dl)
    ds_cp = ce(41, m1, lz1, cl)
    ds_ca = kl(2, m1, lz1, sx1)
    cs_da = kl(41, m2, lz2, sx2)
    cs_cp = ce(84, m3, lz3, cl)

    cos_total = jnp.sum(jnp.where(col == 86, S, 0.0))
    l_r = 1.0 - cos_total * (1.0 / (B * L))

    lane = lax.broadcasted_iota(jnp.int32, (1, 128), 1)
    pre = (jnp.where(lane == 0, ds_dp, 0.0)
           + jnp.where(lane == 1, ds_ca, 0.0)
           + jnp.where(lane == 2, cs_da, 0.0)
           + jnp.where(lane == 3, cs_cp, 0.0))
    sg = jax.nn.sigmoid(pre)
    l_ds = (jnp.sum(jnp.where(lane == 0, sg, 0.0))
            + jnp.sum(jnp.where(lane == 1, sg, 0.0)))
    l_cs = (jnp.sum(jnp.where(lane == 2, sg, 0.0))
            + jnp.sum(jnp.where(lane == 3, sg, 0.0)))
    l_joint = l_ds + l_cs + l_r

    vals = (l_joint, l_ds, ds_dp, ds_cp, ds_ca, l_cs, cs_da, cs_cp, l_r)
    row = jnp.zeros((1, 128), jnp.float32)
    for i, v in enumerate(vals):
        row = row + jnp.where(lane == i, v, 0.0)
    out_ref[...] = row


def kernel(seq_word_ids, deno_labels, cono_labels, pretrained_table,
           deno_table, cono_table, W_deno, b_deno, W_cono, b_cono):
    f32 = jnp.float32
    idx = seq_word_ids.astype(jnp.int32).reshape(_NW, _NCH, _CHT)
    W0 = (jnp.zeros((D, DQ), f32)
          .at[:, 0:41].set(W_deno).at[:, 41:43].set(W_cono))
    W1 = (jnp.zeros((D, DQ), f32)
          .at[:, 43:84].set(W_deno).at[:, 84:86].set(W_cono))

    Q = pl.pallas_call(
        _precompute_body,
        grid=(V // _VB,),
        in_specs=[
            pl.BlockSpec((_VB, D), lambda i: (i, 0)),
            pl.BlockSpec((_VB, D), lambda i: (i, 0)),
            pl.BlockSpec((_VB, D), lambda i: (i, 0)),
            pl.BlockSpec((D, DQ), lambda i: (0, 0)),
            pl.BlockSpec((D, DQ), lambda i: (0, 0)),
        ],
        out_specs=pl.BlockSpec((_VB, DQ), lambda i: (i, 0)),
        out_shape=jax.ShapeDtypeStruct((V, DQ), f32),
    )(deno_table, cono_table, pretrained_table, W0, W1)

    S = _make_gather_sum()(Q, idx)

    bias = (jnp.zeros((1, DS), f32)
            .at[0, 0:41].set(b_deno).at[0, 41:43].set(b_cono)
            .at[0, 43:84].set(b_deno).at[0, 84:86].set(b_cono))
    dl = deno_labels.astype(jnp.int32)[:, None]
    cl = cono_labels.astype(jnp.int32)[:, None]

    res = pl.pallas_call(
        _loss_body,
        out_shape=jax.ShapeDtypeStruct((1, 128), f32),
    )(S, dl, cl, bias)
    return res[0, :9]


# speedup vs baseline: 16.5177x; 1.0173x over previous
import functools
import math

import jax
import jax.numpy as jnp
from jax import lax
from jax.experimental import pallas as pl
from jax.experimental.pallas import tpu as pltpu
from jax.experimental.pallas import tpu_sc as plsc

V = 100000
D = 128
B = 4096
L = 50
ND = 41
NCLS = 2
DQ = 128
DS = 96


_VB = 5000


def _precompute_body(deno_ref, cono_ref, pre_ref, w0_ref, w1_ref, q_ref):
    d = deno_ref[...]
    c = cono_ref[...]
    p = pre_ref[...]
    q = jnp.dot(d, w0_ref[...], preferred_element_type=jnp.float32)
    q = q + jnp.dot(c, w1_ref[...], preferred_element_type=jnp.float32)
    s = d + c
    num = jnp.sum(s * p, axis=1, keepdims=True)
    den = (jnp.sqrt(jnp.sum(s * s, axis=1, keepdims=True))
           * jnp.sqrt(jnp.sum(p * p, axis=1, keepdims=True)) + 1e-8)
    cos = num / den
    colq = lax.broadcasted_iota(jnp.int32, q.shape, 1)
    q_ref[...] = jnp.where(colq == 86, cos, q)


_NW = 32
_BPW = B // _NW
_CHB = 2
_CHT = _CHB * L
_NCH = _BPW // _CHB

@functools.lru_cache(maxsize=1)
def _make_gather_sum():
    mesh = plsc.VectorSubcoreMesh(core_axis_name="c", subcore_axis_name="s")

    nbuf = 4
    njc = DS // 16

    @functools.partial(
        pl.kernel,
        out_type=jax.ShapeDtypeStruct((B, DS), jnp.float32),
        mesh=mesh,
        scratch_types=[
            pltpu.VMEM((_NCH, _CHT), jnp.int32),
        ] + [pltpu.VMEM((_CHT, DQ), jnp.float32)] * nbuf + [
            pltpu.VMEM((_BPW, DS), jnp.float32),
        ] + [pltpu.SemaphoreType.DMA] * nbuf,
    )
    def _gather_sum(q_hbm, idx_hbm, out_hbm, idx_v, *scratch):
        bufs = scratch[:nbuf]
        out_v = scratch[nbuf]
        sems = scratch[nbuf + 1:]
        wid = lax.axis_index("s") * 2 + lax.axis_index("c")
        pltpu.sync_copy(idx_hbm.at[wid], idx_v)

        def fire(c, buf, sem):
            pltpu.async_copy(q_hbm.at[idx_v.at[c]], buf, sem)

        def wait(buf, sem):
            pltpu.make_async_copy(q_hbm.at[idx_v.at[0]], buf, sem).wait()

        def reduce_chunk(c, buf):
            for b in range(_CHB):
                base = b * L

                def lbody(l, accs):
                    return tuple(
                        accs[j] + buf[base + l, pl.ds(j * 16, 16)]
                        for j in range(njc))

                accs = lax.fori_loop(
                    0, L, lbody,
                    tuple(jnp.zeros((16,), jnp.float32)
                          for _ in range(njc)),
                    unroll=10)
                for j in range(njc):
                    out_v[c * _CHB + b, pl.ds(j * 16, 16)] = accs[j]

        for k in range(nbuf):
            fire(k, bufs[k], sems[k])

        nsup = _NCH // nbuf

        def body(s, carry):
            base = nbuf * s
            for k in range(nbuf):
                wait(bufs[k], sems[k])
                reduce_chunk(base + k, bufs[k])

                @pl.when(s < nsup - 1)
                def _(k=k):
                    fire(base + k + nbuf, bufs[k], sems[k])
            return carry

        lax.fori_loop(0, nsup, body, 0)
        pltpu.sync_copy(out_v, out_hbm.at[pl.ds(wid * _BPW, _BPW)])

    return _gather_sum


def _loss_body(s_ref, dl_ref, cl_ref, bias_ref, out_ref):
    S = s_ref[...]
    x = S * (1.0 / L) + bias_ref[...]
    col = lax.broadcasted_iota(jnp.int32, (B, DS), 1)
    dl = dl_ref[...]
    cl = cl_ref[...]

    def grp(lo, hi):
        mask = (col >= lo) & (col < hi)
        xm = jnp.where(mask, x, -1e30)
        m = jnp.max(xm, axis=1, keepdims=True)
        e = jnp.where(mask, jnp.exp(x - m), 0.0)
        logZ = jnp.log(jnp.sum(e, axis=1, keepdims=True))
        sum_x = jnp.sum(jnp.where(mask, x, 0.0))
        return m, logZ, sum_x

    def ce(lo, m, logZ, lab):
        sel = col == (lab + lo)
        x_sel = jnp.sum(jnp.where(sel, x, 0.0))
        return (jnp.sum(m) + jnp.sum(logZ) - x_sel) * (1.0 / B)

    def kl(gsize, m, logZ, sum_x):
        u = 1.0 / gsize
        lu = math.log(u)
        return u * (B * gsize * lu - sum_x
                    + gsize * (jnp.sum(m) + jnp.sum(logZ))) * (1.0 / B)

    m0, lz0, _ = grp(0, 41)
    m1, lz1, sx1 = grp(41, 43)
    m2, lz2, sx2 = grp(43, 84)
    m3, lz3, _ = grp(84, 86)

    ds_dp = ce(0, m0, lz0, dl)
    ds_cp = ce(41, m1, lz1, cl)
    ds_ca = kl(2, m1, lz1, sx1)
    cs_da = kl(41, m2, lz2, sx2)
    cs_cp = ce(84, m3, lz3, cl)

    cos_total = jnp.sum(jnp.where(col == 86, S, 0.0))
    l_r = 1.0 - cos_total * (1.0 / (B * L))

    lane = lax.broadcasted_iota(jnp.int32, (1, 128), 1)
    pre = (jnp.where(lane == 0, ds_dp, 0.0)
           + jnp.where(lane == 1, ds_ca, 0.0)
           + jnp.where(lane == 2, cs_da, 0.0)
           + jnp.where(lane == 3, cs_cp, 0.0))
    sg = jax.nn.sigmoid(pre)
    l_ds = (jnp.sum(jnp.where(lane == 0, sg, 0.0))
            + jnp.sum(jnp.where(lane == 1, sg, 0.0)))
    l_cs = (jnp.sum(jnp.where(lane == 2, sg, 0.0))
            + jnp.sum(jnp.where(lane == 3, sg, 0.0)))
    l_joint = l_ds + l_cs + l_r

    vals = (l_joint, l_ds, ds_dp, ds_cp, ds_ca, l_cs, cs_da, cs_cp, l_r)
    row = jnp.zeros((1, 128), jnp.float32)
    for i, v in enumerate(vals):
        row = row + jnp.where(lane == i, v, 0.0)
    out_ref[...] = row


def kernel(seq_word_ids, deno_labels, cono_labels, pretrained_table,
           deno_table, cono_table, W_deno, b_deno, W_cono, b_cono):
    f32 = jnp.float32
    idx = seq_word_ids.astype(jnp.int32).reshape(_NW, _NCH, _CHT)
    W0 = (jnp.zeros((D, DQ), f32)
          .at[:, 0:41].set(W_deno).at[:, 41:43].set(W_cono))
    W1 = (jnp.zeros((D, DQ), f32)
          .at[:, 43:84].set(W_deno).at[:, 84:86].set(W_cono))

    Q = pl.pallas_call(
        _precompute_body,
        grid=(V // _VB,),
        in_specs=[
            pl.BlockSpec((_VB, D), lambda i: (i, 0)),
            pl.BlockSpec((_VB, D), lambda i: (i, 0)),
            pl.BlockSpec((_VB, D), lambda i: (i, 0)),
            pl.BlockSpec((D, DQ), lambda i: (0, 0)),
            pl.BlockSpec((D, DQ), lambda i: (0, 0)),
        ],
        out_specs=pl.BlockSpec((_VB, DQ), lambda i: (i, 0)),
        out_shape=jax.ShapeDtypeStruct((V, DQ), f32),
    )(deno_table, cono_table, pretrained_table, W0, W1)

    S = _make_gather_sum()(Q, idx)

    bias = (jnp.zeros((1, DS), f32)
            .at[0, 0:41].set(b_deno).at[0, 41:43].set(b_cono)
            .at[0, 43:84].set(b_deno).at[0, 84:86].set(b_cono))
    dl = deno_labels.astype(jnp.int32)[:, None]
    cl = cono_labels.astype(jnp.int32)[:, None]

    res = pl.pallas_call(
        _loss_body,
        out_shape=jax.ShapeDtypeStruct((1, 128), f32),
    )(S, dl, cl, bias)
    return res[0, :9]


# speedup vs baseline: 16.7762x; 1.0156x over previous
import functools
import math

import jax
import jax.numpy as jnp
from jax import lax
from jax.experimental import pallas as pl
from jax.experimental.pallas import tpu as pltpu
from jax.experimental.pallas import tpu_sc as plsc

V = 100000
D = 128
B = 4096
L = 50
ND = 41
NCLS = 2
DQ = 128
DS = 96


_VB = 10000


def _precompute_body(deno_ref, cono_ref, pre_ref, w0_ref, w1_ref, q_ref):
    d = deno_ref[...]
    c = cono_ref[...]
    p = pre_ref[...]
    q = jnp.dot(d, w0_ref[...], preferred_element_type=jnp.float32)
    q = q + jnp.dot(c, w1_ref[...], preferred_element_type=jnp.float32)
    s = d + c
    num = jnp.sum(s * p, axis=1, keepdims=True)
    den = (jnp.sqrt(jnp.sum(s * s, axis=1, keepdims=True))
           * jnp.sqrt(jnp.sum(p * p, axis=1, keepdims=True)) + 1e-8)
    cos = num / den
    colq = lax.broadcasted_iota(jnp.int32, q.shape, 1)
    q_ref[...] = jnp.where(colq == 86, cos, q)


_NW = 32
_BPW = B // _NW
_CHB = 2
_CHT = _CHB * L
_NCH = _BPW // _CHB

@functools.lru_cache(maxsize=1)
def _make_gather_sum():
    mesh = plsc.VectorSubcoreMesh(core_axis_name="c", subcore_axis_name="s")

    nbuf = 4
    njc = DS // 16

    @functools.partial(
        pl.kernel,
        out_type=jax.ShapeDtypeStruct((B, DS), jnp.float32),
        mesh=mesh,
        scratch_types=[
            pltpu.VMEM((_NCH, _CHT), jnp.int32),
        ] + [pltpu.VMEM((_CHT, DQ), jnp.float32)] * nbuf + [
            pltpu.VMEM((_BPW, DS), jnp.float32),
        ] + [pltpu.SemaphoreType.DMA] * nbuf,
    )
    def _gather_sum(q_hbm, idx_hbm, out_hbm, idx_v, *scratch):
        bufs = scratch[:nbuf]
        out_v = scratch[nbuf]
        sems = scratch[nbuf + 1:]
        wid = lax.axis_index("s") * 2 + lax.axis_index("c")
        pltpu.sync_copy(idx_hbm.at[wid], idx_v)

        def fire(c, buf, sem):
            pltpu.async_copy(q_hbm.at[idx_v.at[c]], buf, sem)

        def wait(buf, sem):
            pltpu.make_async_copy(q_hbm.at[idx_v.at[0]], buf, sem).wait()

        def reduce_chunk(c, buf):
            for b in range(_CHB):
                base = b * L

                def lbody(l, accs):
                    return tuple(
                        accs[j] + buf[base + l, pl.ds(j * 16, 16)]
                        for j in range(njc))

                accs = lax.fori_loop(
                    0, L, lbody,
                    tuple(jnp.zeros((16,), jnp.float32)
                          for _ in range(njc)),
                    unroll=10)
                for j in range(njc):
                    out_v[c * _CHB + b, pl.ds(j * 16, 16)] = accs[j]

        for k in range(nbuf):
            fire(k, bufs[k], sems[k])

        nsup = _NCH // nbuf

        def body(s, carry):
            base = nbuf * s
            for k in range(nbuf):
                wait(bufs[k], sems[k])
                reduce_chunk(base + k, bufs[k])

                @pl.when(s < nsup - 1)
                def _(k=k):
                    fire(base + k + nbuf, bufs[k], sems[k])
            return carry

        lax.fori_loop(0, nsup, body, 0)
        pltpu.sync_copy(out_v, out_hbm.at[pl.ds(wid * _BPW, _BPW)])

    return _gather_sum


def _loss_body(s_ref, dl_ref, cl_ref, bias_ref, out_ref):
    S = s_ref[...]
    x = S * (1.0 / L) + bias_ref[...]
    col = lax.broadcasted_iota(jnp.int32, (B, DS), 1)
    dl = dl_ref[...]
    cl = cl_ref[...]

    def grp(lo, hi):
        mask = (col >= lo) & (col < hi)
        xm = jnp.where(mask, x, -1e30)
        m = jnp.max(xm, axis=1, keepdims=True)
        e = jnp.where(mask, jnp.exp(x - m), 0.0)
        logZ = jnp.log(jnp.sum(e, axis=1, keepdims=True))
        sum_x = jnp.sum(jnp.where(mask, x, 0.0))
        return m, logZ, sum_x

    def ce(lo, m, logZ, lab):
        sel = col == (lab + lo)
        x_sel = jnp.sum(jnp.where(sel, x, 0.0))
        return (jnp.sum(m) + jnp.sum(logZ) - x_sel) * (1.0 / B)

    def kl(gsize, m, logZ, sum_x):
        u = 1.0 / gsize
        lu = math.log(u)
        return u * (B * gsize * lu - sum_x
                    + gsize * (jnp.sum(m) + jnp.sum(logZ))) * (1.0 / B)

    m0, lz0, _ = grp(0, 41)
    m1, lz1, sx1 = grp(41, 43)
    m2, lz2, sx2 = grp(43, 84)
    m3, lz3, _ = grp(84, 86)

    ds_dp = ce(0, m0, lz0, dl)
    ds_cp = ce(41, m1, lz1, cl)
    ds_ca = kl(2, m1, lz1, sx1)
    cs_da = kl(41, m2, lz2, sx2)
    cs_cp = ce(84, m3, lz3, cl)

    cos_total = jnp.sum(jnp.where(col == 86, S, 0.0))
    l_r = 1.0 - cos_total * (1.0 / (B * L))

    lane = lax.broadcasted_iota(jnp.int32, (1, 128), 1)
    pre = (jnp.where(lane == 0, ds_dp, 0.0)
           + jnp.where(lane == 1, ds_ca, 0.0)
           + jnp.where(lane == 2, cs_da, 0.0)
           + jnp.where(lane == 3, cs_cp, 0.0))
    sg = jax.nn.sigmoid(pre)
    l_ds = (jnp.sum(jnp.where(lane == 0, sg, 0.0))
            + jnp.sum(jnp.where(lane == 1, sg, 0.0)))
    l_cs = (jnp.sum(jnp.where(lane == 2, sg, 0.0))
            + jnp.sum(jnp.where(lane == 3, sg, 0.0)))
    l_joint = l_ds + l_cs + l_r

    vals = (l_joint, l_ds, ds_dp, ds_cp, ds_ca, l_cs, cs_da, cs_cp, l_r)
    row = jnp.zeros((1, 128), jnp.float32)
    for i, v in enumerate(vals):
        row = row + jnp.where(lane == i, v, 0.0)
    out_ref[...] = row


def kernel(seq_word_ids, deno_labels, cono_labels, pretrained_table,
           deno_table, cono_table, W_deno, b_deno, W_cono, b_cono):
    f32 = jnp.float32
    idx = seq_word_ids.astype(jnp.int32).reshape(_NW, _NCH, _CHT)
    W0 = (jnp.zeros((D, DQ), f32)
          .at[:, 0:41].set(W_deno).at[:, 41:43].set(W_cono))
    W1 = (jnp.zeros((D, DQ), f32)
          .at[:, 43:84].set(W_deno).at[:, 84:86].set(W_cono))

    Q = pl.pallas_call(
        _precompute_body,
        grid=(V // _VB,),
        in_specs=[
            pl.BlockSpec((_VB, D), lambda i: (i, 0)),
            pl.BlockSpec((_VB, D), lambda i: (i, 0)),
            pl.BlockSpec((_VB, D), lambda i: (i, 0)),
            pl.BlockSpec((D, DQ), lambda i: (0, 0)),
            pl.BlockSpec((D, DQ), lambda i: (0, 0)),
        ],
        out_specs=pl.BlockSpec((_VB, DQ), lambda i: (i, 0)),
        out_shape=jax.ShapeDtypeStruct((V, DQ), f32),
    )(deno_table, cono_table, pretrained_table, W0, W1)

    S = _make_gather_sum()(Q, idx)

    bias = (jnp.zeros((1, DS), f32)
            .at[0, 0:41].set(b_deno).at[0, 41:43].set(b_cono)
            .at[0, 43:84].set(b_deno).at[0, 84:86].set(b_cono))
    dl = deno_labels.astype(jnp.int32)[:, None]
    cl = cono_labels.astype(jnp.int32)[:, None]

    res = pl.pallas_call(
        _loss_body,
        out_shape=jax.ShapeDtypeStruct((1, 128), f32),
    )(S, dl, cl, bias)
    return res[0, :9]
